# Initial kernel scaffold; baseline (speedup 1.0000x reference)
import jax
import jax.numpy as jnp
from jax.experimental import pallas as pl


def kernel(H_t, X_t, cond_embedding, edges, edge_types, generate_mask, batch_ids, t, params):
    raise NotImplementedError("write your pallas kernel here")


# speedup vs baseline: 2.6164x; 2.6164x over previous
import functools

import numpy as np
import jax
import jax.numpy as jnp
from jax import lax
from jax.experimental import pallas as pl
from jax.experimental.pallas import tpu as pltpu
from jax.experimental.pallas import tpu_sc as plsc

_N = 10000
_E = 320000
_H = 128
_XW = 16
_NP = 10240
_CH = 128
_NW = 32
_E2 = 323584
_WCHUNK = _E2 // (_NW * _CH)
_TCHUNK = _E2 // (16 * _CH)
_STRIPE = _NP // 16
_BE = 2048
_BN = 2000

_f32 = jnp.float32


def _mesh():
    return plsc.VectorSubcoreMesh(core_axis_name="c", subcore_axis_name="s")


def _gather_body(p_hbm, q_hbm, xp_hbm, src_hbm, dst_hbm,
                 ps_hbm, qd_hbm, xs_hbm, xd_hbm,
                 sidx, didx, prow, qrow, xsrow, xdrow, sem):
    sid = lax.axis_index("s")
    wid = sid * 2 + lax.axis_index("c")

    def body(j, carry):
        off = (wid + j * _NW) * _CH
        pltpu.sync_copy(src_hbm.at[pl.ds(off, _CH)], sidx)
        pltpu.sync_copy(dst_hbm.at[pl.ds(off, _CH)], didx)
        c1 = pltpu.async_copy(p_hbm.at[sidx], prow, sem)
        c2 = pltpu.async_copy(q_hbm.at[didx], qrow, sem)
        c3 = pltpu.async_copy(xp_hbm.at[sidx], xsrow, sem)
        c4 = pltpu.async_copy(xp_hbm.at[didx], xdrow, sem)
        c1.wait(); c2.wait(); c3.wait(); c4.wait()
        pltpu.sync_copy(prow, ps_hbm.at[pl.ds(off, _CH)])
        pltpu.sync_copy(qrow, qd_hbm.at[pl.ds(off, _CH)])
        pltpu.sync_copy(xsrow, xs_hbm.at[pl.ds(off, _CH)])
        pltpu.sync_copy(xdrow, xd_hbm.at[pl.ds(off, _CH)])
        return carry

    lax.fori_loop(0, _WCHUNK, body, 0)


def _sc_gather(P, Q, xp, src, dst):
    return pl.kernel(
        _gather_body,
        out_type=(
            jax.ShapeDtypeStruct((_E2, _H), _f32),
            jax.ShapeDtypeStruct((_E2, _H), _f32),
            jax.ShapeDtypeStruct((_E2, _H), _f32),
            jax.ShapeDtypeStruct((_E2, _H), _f32),
        ),
        mesh=_mesh(),
        scratch_types=[
            pltpu.VMEM((_CH,), jnp.int32),
            pltpu.VMEM((_CH,), jnp.int32),
            pltpu.VMEM((_CH, _H), _f32),
            pltpu.VMEM((_CH, _H), _f32),
            pltpu.VMEM((_CH, _H), _f32),
            pltpu.VMEM((_CH, _H), _f32),
            pltpu.SemaphoreType.DMA,
        ],
    )(P, Q, xp, src, dst)


def _scatter_body(m2_hbm, ox_hbm, dst_hbm, z128_hbm,
                  agg_hbm, xagg_hbm,
                  didx, vrow, acc_sh):
    cid = lax.axis_index("c")
    sid = lax.axis_index("s")
    r0 = sid * _STRIPE
    pltpu.sync_copy(z128_hbm.at[pl.ds(r0, _STRIPE)], acc_sh.at[pl.ds(r0, _STRIPE)])
    plsc.subcore_barrier()

    def body0(j, carry):
        off = (sid + j * 16) * _CH
        pltpu.sync_copy(dst_hbm.at[pl.ds(off, _CH)], didx)
        pltpu.sync_copy(m2_hbm.at[pl.ds(off, _CH)], vrow)
        pltpu.sync_copy(vrow, acc_sh.at[didx], add=True)
        return carry

    def body1(j, carry):
        off = (sid + j * 16) * _CH
        pltpu.sync_copy(dst_hbm.at[pl.ds(off, _CH)], didx)
        pltpu.sync_copy(ox_hbm.at[pl.ds(off, _CH)], vrow)
        pltpu.sync_copy(vrow, acc_sh.at[didx], add=True)
        return carry

    @pl.when(cid == 0)
    def _():
        lax.fori_loop(0, _TCHUNK, body0, 0)

    @pl.when(cid == 1)
    def _():
        lax.fori_loop(0, _TCHUNK, body1, 0)

    plsc.subcore_barrier()

    @pl.when(cid == 0)
    def _():
        pltpu.sync_copy(acc_sh.at[pl.ds(r0, _STRIPE)], agg_hbm.at[pl.ds(r0, _STRIPE)])

    @pl.when(cid == 1)
    def _():
        pltpu.sync_copy(acc_sh.at[pl.ds(r0, _STRIPE)], xagg_hbm.at[pl.ds(r0, _STRIPE)])


def _sc_scatter(m2, ox, dst, z128):
    return pl.kernel(
        _scatter_body,
        out_type=(
            jax.ShapeDtypeStruct((_NP, _H), _f32),
            jax.ShapeDtypeStruct((_NP, _H), _f32),
        ),
        mesh=_mesh(),
        scratch_types=[
            pltpu.VMEM((_CH,), jnp.int32),
            pltpu.VMEM((_CH, _H), _f32),
            pltpu.VMEM_SHARED((_NP, _H), _f32),
        ],
    )(m2, ox, dst, z128)


def _edge_body(ps, qd, xs, xd, et, w1r, tb0, dtb, We2, be2, Wx1, bx1, wx2, bx2,
               m2o, oxo):
    relf = xd[...] - xs[...]
    iot = lax.broadcasted_iota(jnp.int32, relf.shape, 1)
    lm = (iot < 3).astype(_f32)
    c3 = (iot == 3).astype(_f32)
    rel = relf * lm
    d2 = jnp.sum(rel * rel, axis=1, keepdims=True)
    e1 = ps[...] + qd[...] + d2 * w1r[...] + tb0[...] + et[...] * dtb[...]
    m1 = e1 * jax.nn.sigmoid(e1)
    m2 = jnp.dot(m1, We2[...], preferred_element_type=_f32) + be2[...]
    m2 = m2 * jax.nn.sigmoid(m2)
    m3 = jnp.dot(m2, Wx1[...], preferred_element_type=_f32) + bx1[...]
    m3 = m3 * jax.nn.sigmoid(m3)
    w = jnp.sum(m3 * wx2[...], axis=1, keepdims=True) + bx2[0, 0]
    m2o[...] = m2
    oxo[...] = rel * w + c3


def _edge_mlp(ps, qd, xs, xd, et, w1r, tb0, dtb, We2, be2, Wx1, bx1, wx2, bx2):
    nblk = _E2 // _BE
    row = lambda i: (i, 0)
    rep = lambda i: (0, 0)
    return pl.pallas_call(
        _edge_body,
        grid=(nblk,),
        in_specs=[
            pl.BlockSpec((_BE, _H), row),
            pl.BlockSpec((_BE, _H), row),
            pl.BlockSpec((_BE, _H), row),
            pl.BlockSpec((_BE, _H), row),
            pl.BlockSpec((_BE, 1), row),
            pl.BlockSpec((1, _H), rep),
            pl.BlockSpec((1, _H), rep),
            pl.BlockSpec((1, _H), rep),
            pl.BlockSpec((_H, _H), rep),
            pl.BlockSpec((1, _H), rep),
            pl.BlockSpec((_H, _H), rep),
            pl.BlockSpec((1, _H), rep),
            pl.BlockSpec((1, _H), rep),
            pl.BlockSpec(memory_space=pltpu.SMEM),
        ],
        out_specs=[
            pl.BlockSpec((_BE, _H), row),
            pl.BlockSpec((_BE, _H), row),
        ],
        out_shape=[
            jax.ShapeDtypeStruct((_E2, _H), _f32),
            jax.ShapeDtypeStruct((_E2, _H), _f32),
        ],
        compiler_params=pltpu.CompilerParams(
            dimension_semantics=("arbitrary",)),
    )(ps, qd, xs, xd, et, w1r, tb0, dtb, We2, be2, Wx1, bx1, wx2, bx2)


def _node_body(first, last, *refs):
    if last:
        (h, xp, agg, xagg, Wh1a, Wh1b, bh1, Wh2, bh2,
         Wo, bo, mk, vh_o, vx_o) = refs
    else:
        (h, xp, agg, xagg, Wh1a, Wh1b, bh1, Wh2, bh2,
         A1n, B1n, h_o, xp_o, p_o, q_o) = refs
    hv = h[...]
    aggv = agg[...]
    xa = xagg[...]
    xpv = xp[...]
    iot = lax.broadcasted_iota(jnp.int32, xpv.shape, 1)
    lm = (iot < 3).astype(_f32)
    c3 = (iot == 3).astype(_f32)
    if first:
        deg = jnp.maximum(xa[:, 3:4], 1.0)
    else:
        deg = xpv[:, 3:4]
    xnew = (xpv + xa * lm / deg) * lm + deg * c3
    t1 = (jnp.dot(hv, Wh1a[...], preferred_element_type=_f32)
          + jnp.dot(aggv, Wh1b[...], preferred_element_type=_f32) + bh1[...])
    t1 = t1 * jax.nn.sigmoid(t1)
    hnew = hv + jnp.dot(t1, Wh2[...], preferred_element_type=_f32) + bh2[...]
    if last:
        vh = jnp.dot(hnew, Wo[...], preferred_element_type=_f32) + bo[...]
        vh_o[...] = vh * mk[...]
        vx_o[...] = (xnew * mk[...])[:, :_XW]
    else:
        h_o[...] = hnew
        xp_o[...] = xnew
        p_o[...] = jnp.dot(hnew, A1n[...], preferred_element_type=_f32)
        q_o[...] = jnp.dot(hnew, B1n[...], preferred_element_type=_f32)


def _node_update(first, last, h, xp, agg, xagg, Wh1a, Wh1b, bh1, Wh2, bh2,
                 *tail):
    nblk = _N // _BN
    row = lambda i: (i, 0)
    rep = lambda i: (0, 0)
    in_specs = [
        pl.BlockSpec((_BN, _H), row),
        pl.BlockSpec((_BN, _H), row),
        pl.BlockSpec((_BN, _H), row),
        pl.BlockSpec((_BN, _H), row),
        pl.BlockSpec((_H, _H), rep),
        pl.BlockSpec((_H, _H), rep),
        pl.BlockSpec((1, _H), rep),
        pl.BlockSpec((_H, _H), rep),
        pl.BlockSpec((1, _H), rep),
    ]
    if last:
        in_specs += [
            pl.BlockSpec((_H, _H), rep),
            pl.BlockSpec((1, _H), rep),
            pl.BlockSpec((_BN, 1), row),
        ]
        out_specs = [pl.BlockSpec((_BN, _H), row), pl.BlockSpec((_BN, _XW), row)]
        out_shape = [jax.ShapeDtypeStruct((_N, _H), _f32),
                     jax.ShapeDtypeStruct((_N, _XW), _f32)]
    else:
        in_specs += [
            pl.BlockSpec((_H, _H), rep),
            pl.BlockSpec((_H, _H), rep),
        ]
        out_specs = [pl.BlockSpec((_BN, _H), row), pl.BlockSpec((_BN, _H), row),
                     pl.BlockSpec((_BN, _H), row), pl.BlockSpec((_BN, _H), row)]
        out_shape = [jax.ShapeDtypeStruct((_N, _H), _f32),
                     jax.ShapeDtypeStruct((_N, _H), _f32),
                     jax.ShapeDtypeStruct((_N, _H), _f32),
                     jax.ShapeDtypeStruct((_N, _H), _f32)]
    return pl.pallas_call(
        functools.partial(_node_body, first, last),
        grid=(nblk,),
        in_specs=in_specs,
        out_specs=out_specs,
        out_shape=out_shape,
    )(h, xp, agg, xagg, Wh1a, Wh1b, bh1, Wh2, bh2, *tail)


def _pro_body(ht, cd, tt, freq, W1a, W1b, W1c, W1d, bi1, Wi2, bi2, Wi3, bi3,
              A10, B10, h_o, p_o, q_o):
    ang = tt[...] * freq[...]
    s = jnp.sin(ang)
    c = jnp.cos(ang)
    f = (jnp.dot(ht[...], W1a[...], preferred_element_type=_f32)
         + jnp.dot(cd[...], W1b[...], preferred_element_type=_f32)
         + jnp.dot(s, W1c[...], preferred_element_type=_f32)
         + jnp.dot(c, W1d[...], preferred_element_type=_f32) + bi1[...])
    f = jnp.maximum(f, 0.0)
    f = jnp.maximum(jnp.dot(f, Wi2[...], preferred_element_type=_f32) + bi2[...], 0.0)
    h0 = jnp.dot(f, Wi3[...], preferred_element_type=_f32) + bi3[...]
    h_o[...] = h0
    p_o[...] = jnp.dot(h0, A10[...], preferred_element_type=_f32)
    q_o[...] = jnp.dot(h0, B10[...], preferred_element_type=_f32)


def _prologue(H_t, cond, t2, freq, W1a, W1b, W1c, W1d, bi1, Wi2, bi2, Wi3, bi3,
              A10, B10):
    nblk = _N // _BN
    row = lambda i: (i, 0)
    rep = lambda i: (0, 0)
    half = _H // 2
    return pl.pallas_call(
        _pro_body,
        grid=(nblk,),
        in_specs=[
            pl.BlockSpec((_BN, _H), row),
            pl.BlockSpec((_BN, _H), row),
            pl.BlockSpec((_BN, 1), row),
            pl.BlockSpec((1, half), rep),
            pl.BlockSpec((_H, _H), rep),
            pl.BlockSpec((_H, _H), rep),
            pl.BlockSpec((half, _H), rep),
            pl.BlockSpec((half, _H), rep),
            pl.BlockSpec((1, _H), rep),
            pl.BlockSpec((_H, _H), rep),
            pl.BlockSpec((1, _H), rep),
            pl.BlockSpec((_H, _H), rep),
            pl.BlockSpec((1, _H), rep),
            pl.BlockSpec((_H, _H), rep),
            pl.BlockSpec((_H, _H), rep),
        ],
        out_specs=[pl.BlockSpec((_BN, _H), row)] * 3,
        out_shape=[jax.ShapeDtypeStruct((_N, _H), _f32)] * 3,
    )(H_t, cond, t2, freq, W1a, W1b, W1c, W1d, bi1, Wi2, bi2, Wi3, bi3, A10, B10)


def kernel(H_t, X_t, cond_embedding, edges, edge_types, generate_mask,
           batch_ids, t, params):
    p = params
    pad = _E2 - _E
    src = jnp.pad(edges[0], (0, pad))
    dst_g = jnp.pad(edges[1], (0, pad))
    dst_s = jnp.pad(edges[1], (0, pad), constant_values=_NP - 1)
    etf = jnp.pad(edge_types.astype(_f32), (0, pad)).reshape(_E2, 1)
    mk = generate_mask.astype(_f32).reshape(_N, 1)
    t2 = t.reshape(_N, 1)
    xp = jnp.pad(X_t, ((0, 0), (0, _H - 3)))
    half = _H // 2
    freq = jnp.asarray(
        np.exp(-np.log(10000.0) * np.arange(half, dtype=np.float32) / (half - 1))
    ).reshape(1, half)

    Wi1 = p['Wi1']
    W1a, W1b = Wi1[:_H], Wi1[_H:2 * _H]
    W1c, W1d = Wi1[2 * _H:2 * _H + half], Wi1[2 * _H + half:]
    bi1 = p['bi1'].reshape(1, _H)
    bi2 = p['bi2'].reshape(1, _H)
    bi3 = p['bi3'].reshape(1, _H)
    A1 = [p['We1'][l, :_H] for l in range(3)]
    B1 = [p['We1'][l, _H:2 * _H] for l in range(3)]
    w1r = [p['We1'][l, 2 * _H:2 * _H + 1] for l in range(3)]
    Et = [p['edge_table'] @ p['We1'][l, 2 * _H + 1:] for l in range(3)]
    tb0 = [(Et[l][0] + p['be1'][l]).reshape(1, _H) for l in range(3)]
    dtb = [(Et[l][1] - Et[l][0]).reshape(1, _H) for l in range(3)]
    be2 = [p['be2'][l].reshape(1, _H) for l in range(3)]
    bx1 = [p['bx1'][l].reshape(1, _H) for l in range(3)]
    wx2 = [p['Wx2'][l].reshape(1, _H) for l in range(3)]
    bx2 = [p['bx2'][l].reshape(1, 1) for l in range(3)]
    Wh1a = [p['Wh1'][l, :_H] for l in range(3)]
    Wh1b = [p['Wh1'][l, _H:] for l in range(3)]
    bh1 = [p['bh1'][l].reshape(1, _H) for l in range(3)]
    bh2 = [p['bh2'][l].reshape(1, _H) for l in range(3)]
    bo = p['bo'].reshape(1, _H)

    z128 = jnp.zeros((_NP, _H), _f32)

    h, P, Q = _prologue(H_t, cond_embedding, t2, freq, W1a, W1b, W1c, W1d,
                        bi1, p['Wi2'], bi2, p['Wi3'], bi3, A1[0], B1[0])

    for l in range(3):
        ps, qd, xs, xd = _sc_gather(P, Q, xp, src, dst_g)
        m2, ox = _edge_mlp(ps, qd, xs, xd, etf, w1r[l], tb0[l], dtb[l],
                           p['We2'][l], be2[l], p['Wx1'][l], bx1[l],
                           wx2[l], bx2[l])
        agg, xagg = _sc_scatter(m2, ox, dst_s, z128)
        if l < 2:
            h, xp, P, Q = _node_update(
                l == 0, False, h, xp, agg, xagg,
                Wh1a[l], Wh1b[l], bh1[l], p['Wh2'][l], bh2[l],
                A1[l + 1], B1[l + 1])
        else:
            vh, vxp = _node_update(
                False, True, h, xp, agg, xagg,
                Wh1a[l], Wh1b[l], bh1[l], p['Wh2'][l], bh2[l],
                p['Wo'], bo, mk)
    return vh, vxp[:, :3]


# speedup vs baseline: 2.8632x; 1.0943x over previous
import functools

import numpy as np
import jax
import jax.numpy as jnp
from jax import lax
from jax.experimental import pallas as pl
from jax.experimental.pallas import tpu as pltpu
from jax.experimental.pallas import tpu_sc as plsc

_N = 10000
_E = 320000
_H = 128
_XW = 16
_NP = 10240
_CH = 128
_NW = 32
_E2 = 323584
_WCHUNK = _E2 // (_NW * _CH)
_TCHUNK = _E2 // (16 * _CH)
_STRIPE = _NP // 16
_BE = 2048
_BN = 2000

_f32 = jnp.float32


def _mesh():
    return plsc.VectorSubcoreMesh(core_axis_name="c", subcore_axis_name="s")


_GCH = 64
_GC_PER_W = _E2 // (_NW * _GCH)
_GTRIP = (_GC_PER_W - 2) // 3


def _gather_body(p_hbm, q_hbm, xp_hbm, src_hbm, dst_hbm,
                 ps_hbm, qd_hbm, xs_hbm, xd_hbm,
                 sidx, didx, prow, qrow, xsrow, xdrow, gsems, wsems):
    sid = lax.axis_index("s")
    wid = sid * 2 + lax.axis_index("c")
    last = _GC_PER_W - 1

    def off_of(j):
        return (wid + j * _NW) * _GCH

    def fire(b, j):
        off = off_of(j)
        pltpu.sync_copy(src_hbm.at[pl.ds(off, _GCH)], sidx.at[b])
        pltpu.sync_copy(dst_hbm.at[pl.ds(off, _GCH)], didx.at[b])
        pltpu.async_copy(p_hbm.at[sidx.at[b]], prow.at[b], gsems.at[b])
        pltpu.async_copy(q_hbm.at[didx.at[b]], qrow.at[b], gsems.at[b])
        pltpu.async_copy(xp_hbm.at[sidx.at[b]], xsrow.at[b], gsems.at[b])
        pltpu.async_copy(xp_hbm.at[didx.at[b]], xdrow.at[b], gsems.at[b])

    def wait_gather(b):
        pltpu.make_async_copy(p_hbm.at[pl.ds(0, _GCH)], prow.at[b], gsems.at[b]).wait()
        pltpu.make_async_copy(q_hbm.at[pl.ds(0, _GCH)], qrow.at[b], gsems.at[b]).wait()
        pltpu.make_async_copy(xp_hbm.at[pl.ds(0, _GCH)], xsrow.at[b], gsems.at[b]).wait()
        pltpu.make_async_copy(xp_hbm.at[pl.ds(0, _GCH)], xdrow.at[b], gsems.at[b]).wait()

    def fire_wb(b, j):
        off = off_of(j)
        pltpu.async_copy(prow.at[b], ps_hbm.at[pl.ds(off, _GCH)], wsems.at[b])
        pltpu.async_copy(qrow.at[b], qd_hbm.at[pl.ds(off, _GCH)], wsems.at[b])
        pltpu.async_copy(xsrow.at[b], xs_hbm.at[pl.ds(off, _GCH)], wsems.at[b])
        pltpu.async_copy(xdrow.at[b], xd_hbm.at[pl.ds(off, _GCH)], wsems.at[b])

    def wait_wb(b):
        pltpu.make_async_copy(prow.at[b], ps_hbm.at[pl.ds(0, _GCH)], wsems.at[b]).wait()
        pltpu.make_async_copy(qrow.at[b], qd_hbm.at[pl.ds(0, _GCH)], wsems.at[b]).wait()
        pltpu.make_async_copy(xsrow.at[b], xs_hbm.at[pl.ds(0, _GCH)], wsems.at[b]).wait()
        pltpu.make_async_copy(xdrow.at[b], xd_hbm.at[pl.ds(0, _GCH)], wsems.at[b]).wait()

    for b in (0, 1, 2):
        fire(b, b)

    def trip(i, carry):
        for b in (0, 1, 2):
            wait_gather(b)
            fire_wb(b, 3 * i + b)
        for b in (0, 1, 2):
            wait_wb(b)
            fire(b, jnp.minimum(3 * (i + 1) + b, last))
        return carry

    lax.fori_loop(0, _GTRIP, trip, 0)
    for b in (0, 1, 2):
        wait_gather(b)
        fire_wb(b, min(3 * _GTRIP + b, last))
    for b in (0, 1, 2):
        wait_wb(b)


def _sc_gather(P, Q, xp, src, dst):
    return pl.kernel(
        _gather_body,
        out_type=(
            jax.ShapeDtypeStruct((_E2, _H), _f32),
            jax.ShapeDtypeStruct((_E2, _H), _f32),
            jax.ShapeDtypeStruct((_E2, _H), _f32),
            jax.ShapeDtypeStruct((_E2, _H), _f32),
        ),
        mesh=_mesh(),
        scratch_types=[
            pltpu.VMEM((3, _GCH), jnp.int32),
            pltpu.VMEM((3, _GCH), jnp.int32),
            pltpu.VMEM((3, _GCH, _H), _f32),
            pltpu.VMEM((3, _GCH, _H), _f32),
            pltpu.VMEM((3, _GCH, _H), _f32),
            pltpu.VMEM((3, _GCH, _H), _f32),
            pltpu.SemaphoreType.DMA((3,)),
            pltpu.SemaphoreType.DMA((3,)),
        ],
    )(P, Q, xp, src, dst)


_SCH = 64
_SC_PER_T = _E2 // (16 * _SCH)
_SPAIR = _SC_PER_T // 2


def _scatter_body(m2_hbm, ox_hbm, dst_hbm, z128_hbm,
                  agg_hbm, xagg_hbm,
                  didx, vrow, acc_sh, lsems):
    cid = lax.axis_index("c")
    sid = lax.axis_index("s")
    r0 = sid * _STRIPE
    last = _SC_PER_T - 1
    pltpu.sync_copy(z128_hbm.at[pl.ds(r0, _STRIPE)], acc_sh.at[pl.ds(r0, _STRIPE)])
    plsc.subcore_barrier()

    def off_of(j):
        return (sid + j * 16) * _SCH

    def make_loop(val_hbm):
        def fire(b, j):
            off = off_of(j)
            pltpu.async_copy(dst_hbm.at[pl.ds(off, _SCH)], didx.at[b], lsems.at[b])
            pltpu.async_copy(val_hbm.at[pl.ds(off, _SCH)], vrow.at[b], lsems.at[b])

        def wait(b):
            pltpu.make_async_copy(dst_hbm.at[pl.ds(0, _SCH)], didx.at[b], lsems.at[b]).wait()
            pltpu.make_async_copy(val_hbm.at[pl.ds(0, _SCH)], vrow.at[b], lsems.at[b]).wait()

        def run():
            for b in (0, 1):
                fire(b, b)

            def pair(i, carry):
                for b in (0, 1):
                    wait(b)
                    pltpu.sync_copy(vrow.at[b], acc_sh.at[didx.at[b]], add=True)
                    fire(b, jnp.minimum(2 * (i + 1) + b, last))
                return carry

            lax.fori_loop(0, _SPAIR, pair, 0)
            wait(0)
            wait(1)
        return run

    @pl.when(cid == 0)
    def _():
        make_loop(m2_hbm)()

    @pl.when(cid == 1)
    def _():
        make_loop(ox_hbm)()

    plsc.subcore_barrier()

    @pl.when(cid == 0)
    def _():
        pltpu.sync_copy(acc_sh.at[pl.ds(r0, _STRIPE)], agg_hbm.at[pl.ds(r0, _STRIPE)])

    @pl.when(cid == 1)
    def _():
        pltpu.sync_copy(acc_sh.at[pl.ds(r0, _STRIPE)], xagg_hbm.at[pl.ds(r0, _STRIPE)])


def _sc_scatter(m2, ox, dst, z128):
    return pl.kernel(
        _scatter_body,
        out_type=(
            jax.ShapeDtypeStruct((_NP, _H), _f32),
            jax.ShapeDtypeStruct((_NP, _H), _f32),
        ),
        mesh=_mesh(),
        scratch_types=[
            pltpu.VMEM((2, _SCH), jnp.int32),
            pltpu.VMEM((2, _SCH, _H), _f32),
            pltpu.VMEM_SHARED((_NP, _H), _f32),
            pltpu.SemaphoreType.DMA((2,)),
        ],
    )(m2, ox, dst, z128)


def _edge_body(ps, qd, xs, xd, et, w1r, tb0, dtb, We2, be2, Wx1, bx1, wx2, bx2,
               m2o, oxo):
    relf = xd[...] - xs[...]
    iot = lax.broadcasted_iota(jnp.int32, relf.shape, 1)
    lm = (iot < 3).astype(_f32)
    c3 = (iot == 3).astype(_f32)
    rel = relf * lm
    d2 = jnp.sum(rel * rel, axis=1, keepdims=True)
    e1 = ps[...] + qd[...] + d2 * w1r[...] + tb0[...] + et[...] * dtb[...]
    m1 = e1 * jax.nn.sigmoid(e1)
    m2 = jnp.dot(m1, We2[...], preferred_element_type=_f32) + be2[...]
    m2 = m2 * jax.nn.sigmoid(m2)
    m3 = jnp.dot(m2, Wx1[...], preferred_element_type=_f32) + bx1[...]
    m3 = m3 * jax.nn.sigmoid(m3)
    w = jnp.sum(m3 * wx2[...], axis=1, keepdims=True) + bx2[0, 0]
    m2o[...] = m2
    oxo[...] = rel * w + c3


def _edge_mlp(ps, qd, xs, xd, et, w1r, tb0, dtb, We2, be2, Wx1, bx1, wx2, bx2):
    nblk = _E2 // _BE
    row = lambda i: (i, 0)
    rep = lambda i: (0, 0)
    return pl.pallas_call(
        _edge_body,
        grid=(nblk,),
        in_specs=[
            pl.BlockSpec((_BE, _H), row),
            pl.BlockSpec((_BE, _H), row),
            pl.BlockSpec((_BE, _H), row),
            pl.BlockSpec((_BE, _H), row),
            pl.BlockSpec((_BE, 1), row),
            pl.BlockSpec((1, _H), rep),
            pl.BlockSpec((1, _H), rep),
            pl.BlockSpec((1, _H), rep),
            pl.BlockSpec((_H, _H), rep),
            pl.BlockSpec((1, _H), rep),
            pl.BlockSpec((_H, _H), rep),
            pl.BlockSpec((1, _H), rep),
            pl.BlockSpec((1, _H), rep),
            pl.BlockSpec(memory_space=pltpu.SMEM),
        ],
        out_specs=[
            pl.BlockSpec((_BE, _H), row),
            pl.BlockSpec((_BE, _H), row),
        ],
        out_shape=[
            jax.ShapeDtypeStruct((_E2, _H), _f32),
            jax.ShapeDtypeStruct((_E2, _H), _f32),
        ],
        compiler_params=pltpu.CompilerParams(
            dimension_semantics=("arbitrary",)),
    )(ps, qd, xs, xd, et, w1r, tb0, dtb, We2, be2, Wx1, bx1, wx2, bx2)


def _node_body(first, last, *refs):
    if last:
        (h, xp, agg, xagg, Wh1a, Wh1b, bh1, Wh2, bh2,
         Wo, bo, mk, vh_o, vx_o) = refs
    else:
        (h, xp, agg, xagg, Wh1a, Wh1b, bh1, Wh2, bh2,
         A1n, B1n, h_o, xp_o, p_o, q_o) = refs
    hv = h[...]
    aggv = agg[...]
    xa = xagg[...]
    xpv = xp[...]
    iot = lax.broadcasted_iota(jnp.int32, xpv.shape, 1)
    lm = (iot < 3).astype(_f32)
    c3 = (iot == 3).astype(_f32)
    if first:
        deg = jnp.maximum(xa[:, 3:4], 1.0)
    else:
        deg = xpv[:, 3:4]
    xnew = (xpv + xa * lm / deg) * lm + deg * c3
    t1 = (jnp.dot(hv, Wh1a[...], preferred_element_type=_f32)
          + jnp.dot(aggv, Wh1b[...], preferred_element_type=_f32) + bh1[...])
    t1 = t1 * jax.nn.sigmoid(t1)
    hnew = hv + jnp.dot(t1, Wh2[...], preferred_element_type=_f32) + bh2[...]
    if last:
        vh = jnp.dot(hnew, Wo[...], preferred_element_type=_f32) + bo[...]
        vh_o[...] = vh * mk[...]
        vx_o[...] = (xnew * mk[...])[:, :_XW]
    else:
        h_o[...] = hnew
        xp_o[...] = xnew
        p_o[...] = jnp.dot(hnew, A1n[...], preferred_element_type=_f32)
        q_o[...] = jnp.dot(hnew, B1n[...], preferred_element_type=_f32)


def _node_update(first, last, h, xp, agg, xagg, Wh1a, Wh1b, bh1, Wh2, bh2,
                 *tail):
    nblk = _N // _BN
    row = lambda i: (i, 0)
    rep = lambda i: (0, 0)
    in_specs = [
        pl.BlockSpec((_BN, _H), row),
        pl.BlockSpec((_BN, _H), row),
        pl.BlockSpec((_BN, _H), row),
        pl.BlockSpec((_BN, _H), row),
        pl.BlockSpec((_H, _H), rep),
        pl.BlockSpec((_H, _H), rep),
        pl.BlockSpec((1, _H), rep),
        pl.BlockSpec((_H, _H), rep),
        pl.BlockSpec((1, _H), rep),
    ]
    if last:
        in_specs += [
            pl.BlockSpec((_H, _H), rep),
            pl.BlockSpec((1, _H), rep),
            pl.BlockSpec((_BN, 1), row),
        ]
        out_specs = [pl.BlockSpec((_BN, _H), row), pl.BlockSpec((_BN, _XW), row)]
        out_shape = [jax.ShapeDtypeStruct((_N, _H), _f32),
                     jax.ShapeDtypeStruct((_N, _XW), _f32)]
    else:
        in_specs += [
            pl.BlockSpec((_H, _H), rep),
            pl.BlockSpec((_H, _H), rep),
        ]
        out_specs = [pl.BlockSpec((_BN, _H), row), pl.BlockSpec((_BN, _H), row),
                     pl.BlockSpec((_BN, _H), row), pl.BlockSpec((_BN, _H), row)]
        out_shape = [jax.ShapeDtypeStruct((_N, _H), _f32),
                     jax.ShapeDtypeStruct((_N, _H), _f32),
                     jax.ShapeDtypeStruct((_N, _H), _f32),
                     jax.ShapeDtypeStruct((_N, _H), _f32)]
    return pl.pallas_call(
        functools.partial(_node_body, first, last),
        grid=(nblk,),
        in_specs=in_specs,
        out_specs=out_specs,
        out_shape=out_shape,
    )(h, xp, agg, xagg, Wh1a, Wh1b, bh1, Wh2, bh2, *tail)


def _pro_body(ht, cd, tt, freq, W1a, W1b, W1c, W1d, bi1, Wi2, bi2, Wi3, bi3,
              A10, B10, h_o, p_o, q_o):
    ang = tt[...] * freq[...]
    s = jnp.sin(ang)
    c = jnp.cos(ang)
    f = (jnp.dot(ht[...], W1a[...], preferred_element_type=_f32)
         + jnp.dot(cd[...], W1b[...], preferred_element_type=_f32)
         + jnp.dot(s, W1c[...], preferred_element_type=_f32)
         + jnp.dot(c, W1d[...], preferred_element_type=_f32) + bi1[...])
    f = jnp.maximum(f, 0.0)
    f = jnp.maximum(jnp.dot(f, Wi2[...], preferred_element_type=_f32) + bi2[...], 0.0)
    h0 = jnp.dot(f, Wi3[...], preferred_element_type=_f32) + bi3[...]
    h_o[...] = h0
    p_o[...] = jnp.dot(h0, A10[...], preferred_element_type=_f32)
    q_o[...] = jnp.dot(h0, B10[...], preferred_element_type=_f32)


def _prologue(H_t, cond, t2, freq, W1a, W1b, W1c, W1d, bi1, Wi2, bi2, Wi3, bi3,
              A10, B10):
    nblk = _N // _BN
    row = lambda i: (i, 0)
    rep = lambda i: (0, 0)
    half = _H // 2
    return pl.pallas_call(
        _pro_body,
        grid=(nblk,),
        in_specs=[
            pl.BlockSpec((_BN, _H), row),
            pl.BlockSpec((_BN, _H), row),
            pl.BlockSpec((_BN, 1), row),
            pl.BlockSpec((1, half), rep),
            pl.BlockSpec((_H, _H), rep),
            pl.BlockSpec((_H, _H), rep),
            pl.BlockSpec((half, _H), rep),
            pl.BlockSpec((half, _H), rep),
            pl.BlockSpec((1, _H), rep),
            pl.BlockSpec((_H, _H), rep),
            pl.BlockSpec((1, _H), rep),
            pl.BlockSpec((_H, _H), rep),
            pl.BlockSpec((1, _H), rep),
            pl.BlockSpec((_H, _H), rep),
            pl.BlockSpec((_H, _H), rep),
        ],
        out_specs=[pl.BlockSpec((_BN, _H), row)] * 3,
        out_shape=[jax.ShapeDtypeStruct((_N, _H), _f32)] * 3,
    )(H_t, cond, t2, freq, W1a, W1b, W1c, W1d, bi1, Wi2, bi2, Wi3, bi3, A10, B10)


def kernel(H_t, X_t, cond_embedding, edges, edge_types, generate_mask,
           batch_ids, t, params):
    p = params
    pad = _E2 - _E
    src = jnp.pad(edges[0], (0, pad))
    dst_g = jnp.pad(edges[1], (0, pad))
    dst_s = jnp.pad(edges[1], (0, pad), constant_values=_NP - 1)
    etf = jnp.pad(edge_types.astype(_f32), (0, pad)).reshape(_E2, 1)
    mk = generate_mask.astype(_f32).reshape(_N, 1)
    t2 = t.reshape(_N, 1)
    xp = jnp.pad(X_t, ((0, 0), (0, _H - 3)))
    half = _H // 2
    freq = jnp.asarray(
        np.exp(-np.log(10000.0) * np.arange(half, dtype=np.float32) / (half - 1))
    ).reshape(1, half)

    Wi1 = p['Wi1']
    W1a, W1b = Wi1[:_H], Wi1[_H:2 * _H]
    W1c, W1d = Wi1[2 * _H:2 * _H + half], Wi1[2 * _H + half:]
    bi1 = p['bi1'].reshape(1, _H)
    bi2 = p['bi2'].reshape(1, _H)
    bi3 = p['bi3'].reshape(1, _H)
    A1 = [p['We1'][l, :_H] for l in range(3)]
    B1 = [p['We1'][l, _H:2 * _H] for l in range(3)]
    w1r = [p['We1'][l, 2 * _H:2 * _H + 1] for l in range(3)]
    Et = [p['edge_table'] @ p['We1'][l, 2 * _H + 1:] for l in range(3)]
    tb0 = [(Et[l][0] + p['be1'][l]).reshape(1, _H) for l in range(3)]
    dtb = [(Et[l][1] - Et[l][0]).reshape(1, _H) for l in range(3)]
    be2 = [p['be2'][l].reshape(1, _H) for l in range(3)]
    bx1 = [p['bx1'][l].reshape(1, _H) for l in range(3)]
    wx2 = [p['Wx2'][l].reshape(1, _H) for l in range(3)]
    bx2 = [p['bx2'][l].reshape(1, 1) for l in range(3)]
    Wh1a = [p['Wh1'][l, :_H] for l in range(3)]
    Wh1b = [p['Wh1'][l, _H:] for l in range(3)]
    bh1 = [p['bh1'][l].reshape(1, _H) for l in range(3)]
    bh2 = [p['bh2'][l].reshape(1, _H) for l in range(3)]
    bo = p['bo'].reshape(1, _H)

    z128 = jnp.zeros((_NP, _H), _f32)

    h, P, Q = _prologue(H_t, cond_embedding, t2, freq, W1a, W1b, W1c, W1d,
                        bi1, p['Wi2'], bi2, p['Wi3'], bi3, A1[0], B1[0])

    for l in range(3):
        ps, qd, xs, xd = _sc_gather(P, Q, xp, src, dst_g)
        m2, ox = _edge_mlp(ps, qd, xs, xd, etf, w1r[l], tb0[l], dtb[l],
                           p['We2'][l], be2[l], p['Wx1'][l], bx1[l],
                           wx2[l], bx2[l])
        agg, xagg = _sc_scatter(m2, ox, dst_s, z128)
        if l < 2:
            h, xp, P, Q = _node_update(
                l == 0, False, h, xp, agg, xagg,
                Wh1a[l], Wh1b[l], bh1[l], p['Wh2'][l], bh2[l],
                A1[l + 1], B1[l + 1])
        else:
            vh, vxp = _node_update(
                False, True, h, xp, agg, xagg,
                Wh1a[l], Wh1b[l], bh1[l], p['Wh2'][l], bh2[l],
                p['Wo'], bo, mk)
    return vh, vxp[:, :3]
